# Initial kernel scaffold; baseline (speedup 1.0000x reference)
import jax
import jax.numpy as jnp
from jax.experimental import pallas as pl


def kernel(x_user, x_item, edge_index_user_item, edge_index_item_user, params):
    raise NotImplementedError("write your pallas kernel here")


# speedup vs baseline: 18.4355x; 18.4355x over previous
import functools

import jax
import jax.numpy as jnp
from jax import lax
from jax.experimental import pallas as pl
from jax.experimental.pallas import tpu as pltpu
from jax.experimental.pallas import tpu_sc as plsc

HID = 128
HEADS = 8
DH = HID // HEADS
NNODE = 10000
NEDGE = 160000
ROW_W = HID + 16

NTILES = 16
EPT = NEDGE // NTILES
BATCH = 40
NBATCH = EPT // BATCH
NPAD = 10240
RPT = NPAD // NTILES


_ROWB = 1000
_NROWB = NNODE // _ROWB


def _proj_body(x_ref, w_ref, b_ref, o_ref):
    o_ref[...] = (
        jnp.dot(x_ref[0], w_ref[0], preferred_element_type=jnp.float32)
        + b_ref[0]
    )[None]


def _proj(x_stack, w_stack, b_stack):
    return pl.pallas_call(
        _proj_body,
        out_shape=jax.ShapeDtypeStruct((2, NNODE, 3 * HID), jnp.float32),
        grid=(2, _NROWB),
        in_specs=[
            pl.BlockSpec((1, _ROWB, HID), lambda i, j: (i, j, 0)),
            pl.BlockSpec((1, HID, 3 * HID), lambda i, j: (i, 0, 0)),
            pl.BlockSpec((1, 1, 3 * HID), lambda i, j: (i, 0, 0)),
        ],
        out_specs=pl.BlockSpec((1, _ROWB, 3 * HID), lambda i, j: (i, j, 0)),
    )(x_stack, w_stack, b_stack)


def _out_body(acc_ref, x_ref, w_ref, b_ref, g_ref, s_ref, o_ref):
    acc = acc_ref[0]
    ew = acc[:, :HID]
    sums = acc[:, HID:]
    s_b = jnp.dot(sums, s_ref[...], preferred_element_type=jnp.float32)
    agg = ew / (s_b + 1e-16)
    o = jax.nn.gelu(agg)
    o = jnp.dot(o, w_ref[0], preferred_element_type=jnp.float32) + b_ref[0]
    gate = jax.nn.sigmoid(g_ref[0, 0, 0])
    o_ref[...] = jax.nn.relu(gate * o + (1.0 - gate) * x_ref[0])[None]


def _out_proj(acc_stack, x_stack, w_stack, b_stack, gate_stack, s_const):
    return pl.pallas_call(
        _out_body,
        out_shape=jax.ShapeDtypeStruct((2, NNODE, HID), jnp.float32),
        grid=(2, _NROWB),
        in_specs=[
            pl.BlockSpec((1, _ROWB, ROW_W), lambda i, j: (i, j, 0)),
            pl.BlockSpec((1, _ROWB, HID), lambda i, j: (i, j, 0)),
            pl.BlockSpec((1, HID, HID), lambda i, j: (i, 0, 0)),
            pl.BlockSpec((1, 1, HID), lambda i, j: (i, 0, 0)),
            pl.BlockSpec((1, 1, 1), lambda i, j: (i, 0, 0)),
            pl.BlockSpec((DH, HID), lambda i, j: (0, 0)),
        ],
        out_specs=pl.BlockSpec((1, _ROWB, HID), lambda i, j: (i, j, 0)),
    )(acc_stack, x_stack, w_stack, b_stack, gate_stack, s_const)


def _edge_phase(sid, kk_hbm, q_hbm, vv_hbm, si_hbm, di_hbm, zr_hbm, out_hbm,
                acc, si_v, di_v, kk_v, q_v, vv_v, ew_v, sem0, sem1, sem2):
    pltpu.sync_copy(zr_hbm, acc.at[pl.ds(sid * RPT, RPT)])
    plsc.subcore_barrier()

    lane = lax.iota(jnp.int32, DH)
    base = sid * EPT

    def batch_body(i, carry):
        off = base + i * BATCH
        pltpu.sync_copy(si_hbm.at[pl.ds(off, BATCH)], si_v)
        pltpu.sync_copy(di_hbm.at[pl.ds(off, BATCH)], di_v)
        cpk = pltpu.async_copy(kk_hbm.at[si_v], kk_v, sem0)
        cpq = pltpu.async_copy(q_hbm.at[di_v], q_v, sem1)
        cpv = pltpu.async_copy(vv_hbm.at[si_v], vv_v, sem2)
        cpk.wait()
        cpq.wait()
        cpv.wait()

        def edge_body(e, carry2):
            packed = jnp.zeros((DH,), jnp.float32)
            for h in range(HEADS):
                sl = pl.ds(h * DH, DH)
                a_h = jnp.sum(kk_v[e, sl] * q_v[e, sl])
                esp = jnp.exp(jnp.broadcast_to(a_h, (DH,)))
                packed = jnp.where(lane == h, esp, packed)
                ew_v[e, sl] = vv_v[e, sl] * esp
            ew_v[e, pl.ds(HID, DH)] = packed
            return carry2

        lax.fori_loop(0, BATCH, edge_body, 0)
        pltpu.sync_copy(ew_v, acc.at[di_v], add=True)
        return carry

    lax.fori_loop(0, NBATCH, batch_body, 0)
    plsc.subcore_barrier()
    sl = pl.ds(sid * RPT, RPT)
    pltpu.sync_copy(acc.at[sl], out_hbm.at[sl])


def _sc_body(kk_a, q_a, vv_a, si_a, di_a,
             kk_b, q_b, vv_b, si_b, di_b, zr,
             out_a, out_b,
             acc, si_v, di_v, kk_v, q_v, vv_v, ew_v, sem0, sem1, sem2):
    cid = lax.axis_index("c")
    sid = lax.axis_index("s")

    @pl.when(cid == 0)
    def _():
        _edge_phase(sid, kk_a, q_a, vv_a, si_a, di_a, zr, out_a,
                    acc, si_v, di_v, kk_v, q_v, vv_v, ew_v, sem0, sem1, sem2)

    @pl.when(cid == 1)
    def _():
        _edge_phase(sid, kk_b, q_b, vv_b, si_b, di_b, zr, out_b,
                    acc, si_v, di_v, kk_v, q_v, vv_v, ew_v, sem0, sem1, sem2)


@functools.lru_cache(maxsize=1)
def _build_sc_kernel():
    return pl.kernel(
        _sc_body,
        out_type=(
            jax.ShapeDtypeStruct((NPAD, ROW_W), jnp.float32),
            jax.ShapeDtypeStruct((NPAD, ROW_W), jnp.float32),
        ),
        mesh=plsc.VectorSubcoreMesh(
            core_axis_name="c", subcore_axis_name="s",
            num_cores=2, num_subcores=NTILES),
        scratch_types=[
            pltpu.VMEM_SHARED((NPAD, ROW_W), jnp.float32),
            pltpu.VMEM((BATCH,), jnp.int32),
            pltpu.VMEM((BATCH,), jnp.int32),
            pltpu.VMEM((BATCH, HID), jnp.float32),
            pltpu.VMEM((BATCH, HID), jnp.float32),
            pltpu.VMEM((BATCH, HID), jnp.float32),
            pltpu.VMEM((BATCH, ROW_W), jnp.float32),
            pltpu.SemaphoreType.DMA,
            pltpu.SemaphoreType.DMA,
            pltpu.SemaphoreType.DMA,
        ],
        compiler_params=pltpu.CompilerParams(
            use_tc_tiling_on_sc=False, needs_layout_passes=False),
    )


def _sc_edges(*args):
    return _build_sc_kernel()(*args)


def _block_diag8(rel):
    out = jnp.zeros((HID, HID), jnp.float32)
    for h in range(HEADS):
        out = out.at[h * DH:(h + 1) * DH, h * DH:(h + 1) * DH].set(rel[h])
    return out


def kernel(x_user, x_item, edge_index_user_item, edge_index_item_user, params):
    ei_a = edge_index_user_item.astype(jnp.int32)
    ei_b = edge_index_item_user.astype(jnp.int32)
    si_a, di_a = ei_a[0], ei_a[1]
    si_b, di_b = ei_b[0], ei_b[1]

    s_const = jnp.repeat(jnp.eye(DH, HEADS, dtype=jnp.float32), DH, axis=1)
    zr = jnp.zeros((RPT, ROW_W), jnp.float32)

    x = jnp.stack([x_user, x_item])
    ek = {"user": "user__to__item", "item": "item__to__user"}

    for p in params:
        w_list, b_list, wo_list, bo_list, g_list = [], [], [], [], []
        for t in ("user", "item"):
            rel = ek[t]
            a_bd = _block_diag8(
                p["a_rel"][rel] * (p["p_rel"][rel] / jnp.sqrt(float(DH)))[:, None, None])
            m_bd = _block_diag8(p["m_rel"][rel])
            wk = p["k"][t]["w"] @ a_bd
            bk = p["k"][t]["b"] @ a_bd
            wv = p["v"][t]["w"] @ m_bd
            bv = p["v"][t]["b"] @ m_bd
            w_list.append(jnp.concatenate(
                [wk, p["q"][t]["w"], wv], axis=1))
            b_list.append(jnp.concatenate(
                [bk, p["q"][t]["b"], bv])[None])
            wo_list.append(p["out"][t]["w"])
            bo_list.append(p["out"][t]["b"][None])
            g_list.append(jnp.full((1, 1), p["skip"][t]))

        proj = _proj(x, jnp.stack(w_list), jnp.stack(b_list))
        kk = proj[:, :, :HID]
        q = proj[:, :, HID:2 * HID]
        vv = proj[:, :, 2 * HID:]

        acc_item, acc_user = _sc_edges(
            kk[0], q[1], vv[0], si_a, di_a,
            kk[1], q[0], vv[1], si_b, di_b, zr)
        acc_item = acc_item[:NNODE]
        acc_user = acc_user[:NNODE]

        x = _out_proj(
            jnp.stack([acc_user, acc_item]), x,
            jnp.stack(wo_list), jnp.stack(bo_list), jnp.stack(g_list),
            s_const)

    return (x[0], x[1])


# speedup vs baseline: 20.8024x; 1.1284x over previous
import functools

import jax
import jax.numpy as jnp
from jax import lax
from jax.experimental import pallas as pl
from jax.experimental.pallas import tpu as pltpu
from jax.experimental.pallas import tpu_sc as plsc

HID = 128
HEADS = 8
DH = HID // HEADS
NNODE = 10000
NEDGE = 160000
ROW_W = HID + 16

NTILES = 16
EPT = NEDGE // NTILES
BATCH = 40
NBATCH = EPT // BATCH
NPAD = 10112
RPT = NPAD // NTILES
UNROLL = 8


_ROWB = 1000
_NROWB = NNODE // _ROWB


def _proj_body(x_ref, w_ref, b_ref, o_ref):
    o_ref[...] = (
        jnp.dot(x_ref[0], w_ref[0], preferred_element_type=jnp.float32)
        + b_ref[0]
    )[None]


def _proj(x_stack, w_stack, b_stack):
    return pl.pallas_call(
        _proj_body,
        out_shape=jax.ShapeDtypeStruct((2, NNODE, 3 * HID), jnp.float32),
        grid=(2, _NROWB),
        in_specs=[
            pl.BlockSpec((1, _ROWB, HID), lambda i, j: (i, j, 0)),
            pl.BlockSpec((1, HID, 3 * HID), lambda i, j: (i, 0, 0)),
            pl.BlockSpec((1, 1, 3 * HID), lambda i, j: (i, 0, 0)),
        ],
        out_specs=pl.BlockSpec((1, _ROWB, 3 * HID), lambda i, j: (i, j, 0)),
    )(x_stack, w_stack, b_stack)


def _out_body(acc_ref, x_ref, w_ref, b_ref, g_ref, s_ref, o_ref):
    acc = acc_ref[0]
    ew = acc[:, :HID]
    sums = acc[:, HID:]
    s_b = jnp.dot(sums, s_ref[...], preferred_element_type=jnp.float32)
    agg = ew / (s_b + 1e-16)
    o = jax.nn.gelu(agg)
    o = jnp.dot(o, w_ref[0], preferred_element_type=jnp.float32) + b_ref[0]
    gate = jax.nn.sigmoid(g_ref[0, 0, 0])
    o_ref[...] = jax.nn.relu(gate * o + (1.0 - gate) * x_ref[0])[None]


def _out_proj(acc_stack, x_stack, w_stack, b_stack, gate_stack, s_const):
    return pl.pallas_call(
        _out_body,
        out_shape=jax.ShapeDtypeStruct((2, NNODE, HID), jnp.float32),
        grid=(2, _NROWB),
        in_specs=[
            pl.BlockSpec((1, _ROWB, ROW_W), lambda i, j: (i, j, 0)),
            pl.BlockSpec((1, _ROWB, HID), lambda i, j: (i, j, 0)),
            pl.BlockSpec((1, HID, HID), lambda i, j: (i, 0, 0)),
            pl.BlockSpec((1, 1, HID), lambda i, j: (i, 0, 0)),
            pl.BlockSpec((1, 1, 1), lambda i, j: (i, 0, 0)),
            pl.BlockSpec((DH, HID), lambda i, j: (0, 0)),
        ],
        out_specs=pl.BlockSpec((1, _ROWB, HID), lambda i, j: (i, j, 0)),
    )(acc_stack, x_stack, w_stack, b_stack, gate_stack, s_const)


def _edge_phase(sid, kk_hbm, q_hbm, vv_hbm, si_hbm, di_hbm, zr_hbm, out_hbm,
                acc, bufs, ew_v, sems):
    pltpu.sync_copy(zr_hbm, acc.at[pl.ds(sid * RPT, RPT)])
    plsc.subcore_barrier()

    lane = lax.iota(jnp.int32, DH)
    base = sid * EPT

    def load_idx(b, off):
        si_v, di_v = bufs[b][0], bufs[b][1]
        pltpu.sync_copy(si_hbm.at[pl.ds(off, BATCH)], si_v)
        pltpu.sync_copy(di_hbm.at[pl.ds(off, BATCH)], di_v)

    def start_gathers(b):
        si_v, di_v, kk_v, q_v, vv_v = bufs[b]
        pltpu.async_copy(kk_hbm.at[si_v], kk_v, sems[b][0])
        pltpu.async_copy(q_hbm.at[di_v], q_v, sems[b][1])
        pltpu.async_copy(vv_hbm.at[si_v], vv_v, sems[b][2])

    def wait_gathers(b):
        si_v, di_v, kk_v, q_v, vv_v = bufs[b]
        pltpu.make_async_copy(kk_hbm.at[si_v], kk_v, sems[b][0]).wait()
        pltpu.make_async_copy(q_hbm.at[di_v], q_v, sems[b][1]).wait()
        pltpu.make_async_copy(vv_hbm.at[si_v], vv_v, sems[b][2]).wait()

    def compute_scatter(b):
        si_v, di_v, kk_v, q_v, vv_v = bufs[b]
        wait_gathers(b)

        def step_body(j, carry2):
            for u in range(UNROLL):
                e = j * UNROLL + u
                packed = jnp.zeros((DH,), jnp.float32)
                for h in range(HEADS):
                    sl = pl.ds(h * DH, DH)
                    a_h = jnp.sum(kk_v[e, sl] * q_v[e, sl])
                    esp = jnp.exp(jnp.broadcast_to(a_h, (DH,)))
                    packed = jnp.where(lane == h, esp, packed)
                    ew_v[e, sl] = vv_v[e, sl] * esp
                ew_v[e, pl.ds(HID, DH)] = packed
            return carry2

        lax.fori_loop(0, BATCH // UNROLL, step_body, 0)
        pltpu.sync_copy(ew_v, acc.at[di_v], add=True)

    load_idx(0, base)
    start_gathers(0)

    def pair_body(i, carry):
        off = base + (2 * i) * BATCH
        load_idx(1, off + BATCH)
        start_gathers(1)
        compute_scatter(0)

        @pl.when(i < NBATCH // 2 - 1)
        def _():
            load_idx(0, off + 2 * BATCH)
            start_gathers(0)

        compute_scatter(1)
        return carry

    lax.fori_loop(0, NBATCH // 2, pair_body, 0)
    plsc.subcore_barrier()
    sl = pl.ds(sid * RPT, RPT)
    pltpu.sync_copy(acc.at[sl], out_hbm.at[sl])


def _sc_body(kk_a, q_a, vv_a, si_a, di_a,
             kk_b, q_b, vv_b, si_b, di_b, zr,
             out_a, out_b,
             acc, si0, di0, kk0, q0, vv0, si1, di1, kk1, q1, vv1, ew_v,
             s00, s01, s02, s10, s11, s12):
    cid = lax.axis_index("c")
    sid = lax.axis_index("s")
    bufs = ((si0, di0, kk0, q0, vv0), (si1, di1, kk1, q1, vv1))
    sems = ((s00, s01, s02), (s10, s11, s12))

    @pl.when(cid == 0)
    def _():
        _edge_phase(sid, kk_a, q_a, vv_a, si_a, di_a, zr, out_a,
                    acc, bufs, ew_v, sems)

    @pl.when(cid == 1)
    def _():
        _edge_phase(sid, kk_b, q_b, vv_b, si_b, di_b, zr, out_b,
                    acc, bufs, ew_v, sems)


@functools.lru_cache(maxsize=1)
def _build_sc_kernel():
    return pl.kernel(
        _sc_body,
        out_type=(
            jax.ShapeDtypeStruct((NPAD, ROW_W), jnp.float32),
            jax.ShapeDtypeStruct((NPAD, ROW_W), jnp.float32),
        ),
        mesh=plsc.VectorSubcoreMesh(
            core_axis_name="c", subcore_axis_name="s",
            num_cores=2, num_subcores=NTILES),
        scratch_types=[
            pltpu.VMEM_SHARED((NPAD, ROW_W), jnp.float32),
            pltpu.VMEM((BATCH,), jnp.int32),
            pltpu.VMEM((BATCH,), jnp.int32),
            pltpu.VMEM((BATCH, HID), jnp.float32),
            pltpu.VMEM((BATCH, HID), jnp.float32),
            pltpu.VMEM((BATCH, HID), jnp.float32),
            pltpu.VMEM((BATCH,), jnp.int32),
            pltpu.VMEM((BATCH,), jnp.int32),
            pltpu.VMEM((BATCH, HID), jnp.float32),
            pltpu.VMEM((BATCH, HID), jnp.float32),
            pltpu.VMEM((BATCH, HID), jnp.float32),
            pltpu.VMEM((BATCH, ROW_W), jnp.float32),
            pltpu.SemaphoreType.DMA,
            pltpu.SemaphoreType.DMA,
            pltpu.SemaphoreType.DMA,
            pltpu.SemaphoreType.DMA,
            pltpu.SemaphoreType.DMA,
            pltpu.SemaphoreType.DMA,
        ],
        compiler_params=pltpu.CompilerParams(
            use_tc_tiling_on_sc=False, needs_layout_passes=False),
    )


def _sc_edges(*args):
    return _build_sc_kernel()(*args)


def _block_diag8(rel):
    out = jnp.zeros((HID, HID), jnp.float32)
    for h in range(HEADS):
        out = out.at[h * DH:(h + 1) * DH, h * DH:(h + 1) * DH].set(rel[h])
    return out


def kernel(x_user, x_item, edge_index_user_item, edge_index_item_user, params):
    ei_a = edge_index_user_item.astype(jnp.int32)
    ei_b = edge_index_item_user.astype(jnp.int32)
    si_a, di_a = ei_a[0], ei_a[1]
    si_b, di_b = ei_b[0], ei_b[1]

    s_const = jnp.repeat(jnp.eye(DH, HEADS, dtype=jnp.float32), DH, axis=1)
    zr = jnp.zeros((RPT, ROW_W), jnp.float32)

    x = jnp.stack([x_user, x_item])
    ek = {"user": "user__to__item", "item": "item__to__user"}

    for p in params:
        w_list, b_list, wo_list, bo_list, g_list = [], [], [], [], []
        for t in ("user", "item"):
            rel = ek[t]
            a_bd = _block_diag8(
                p["a_rel"][rel] * (p["p_rel"][rel] / jnp.sqrt(float(DH)))[:, None, None])
            m_bd = _block_diag8(p["m_rel"][rel])
            wk = p["k"][t]["w"] @ a_bd
            bk = p["k"][t]["b"] @ a_bd
            wv = p["v"][t]["w"] @ m_bd
            bv = p["v"][t]["b"] @ m_bd
            w_list.append(jnp.concatenate(
                [wk, p["q"][t]["w"], wv], axis=1))
            b_list.append(jnp.concatenate(
                [bk, p["q"][t]["b"], bv])[None])
            wo_list.append(p["out"][t]["w"])
            bo_list.append(p["out"][t]["b"][None])
            g_list.append(jnp.full((1, 1), p["skip"][t]))

        proj = _proj(x, jnp.stack(w_list), jnp.stack(b_list))
        kk = proj[:, :, :HID]
        q = proj[:, :, HID:2 * HID]
        vv = proj[:, :, 2 * HID:]

        acc_item, acc_user = _sc_edges(
            kk[0], q[1], vv[0], si_a, di_a,
            kk[1], q[0], vv[1], si_b, di_b, zr)
        acc_item = acc_item[:NNODE]
        acc_user = acc_user[:NNODE]

        x = _out_proj(
            jnp.stack([acc_user, acc_item]), x,
            jnp.stack(wo_list), jnp.stack(bo_list), jnp.stack(g_list),
            s_const)

    return (x[0], x[1])


# speedup vs baseline: 51.7621x; 2.4883x over previous
import functools

import jax
import jax.numpy as jnp
from jax import lax
from jax.experimental import pallas as pl
from jax.experimental.pallas import tpu as pltpu
from jax.experimental.pallas import tpu_sc as plsc

HID = 128
HEADS = 8
DH = HID // HEADS
NNODE = 10000
NEDGE = 160000
ROW_W = HID + 16

NTILES = 16
EPT = NEDGE // NTILES
BATCH = 40
NBATCH = EPT // BATCH
NPAD = 10112
RPT = NPAD // NTILES
UNROLL = 8


_ROWB = 1000
_NROWB = NNODE // _ROWB


def _proj_body(x_ref, w_ref, b_ref, o_ref):
    o_ref[...] = (
        jnp.dot(x_ref[0], w_ref[0], preferred_element_type=jnp.float32)
        + b_ref[0]
    )[None]


def _proj(x_stack, w_stack, b_stack):
    return pl.pallas_call(
        _proj_body,
        out_shape=jax.ShapeDtypeStruct((2, NNODE, 3 * HID), jnp.float32),
        grid=(2, _NROWB),
        in_specs=[
            pl.BlockSpec((1, _ROWB, HID), lambda i, j: (i, j, 0)),
            pl.BlockSpec((1, HID, 3 * HID), lambda i, j: (i, 0, 0)),
            pl.BlockSpec((1, 1, 3 * HID), lambda i, j: (i, 0, 0)),
        ],
        out_specs=pl.BlockSpec((1, _ROWB, 3 * HID), lambda i, j: (i, j, 0)),
    )(x_stack, w_stack, b_stack)


def _out_body(acc_ref, e_ref, x_ref, w_ref, b_ref, g_ref, s_ref, o_ref):
    ew = acc_ref[0]
    sums = e_ref[0]
    s_b = jnp.dot(sums, s_ref[...], preferred_element_type=jnp.float32)
    agg = ew / (s_b + 1e-16)
    o = jax.nn.gelu(agg)
    o = jnp.dot(o, w_ref[0], preferred_element_type=jnp.float32) + b_ref[0]
    gate = jax.nn.sigmoid(g_ref[0, 0, 0])
    o_ref[...] = jax.nn.relu(gate * o + (1.0 - gate) * x_ref[0])[None]


def _out_proj(acc_stack, e_stack, x_stack, w_stack, b_stack, gate_stack,
              s_const):
    return pl.pallas_call(
        _out_body,
        out_shape=jax.ShapeDtypeStruct((2, NNODE, HID), jnp.float32),
        grid=(2, _NROWB),
        in_specs=[
            pl.BlockSpec((1, _ROWB, HID), lambda i, j: (i, j, 0)),
            pl.BlockSpec((1, _ROWB, DH), lambda i, j: (i, j, 0)),
            pl.BlockSpec((1, _ROWB, HID), lambda i, j: (i, j, 0)),
            pl.BlockSpec((1, HID, HID), lambda i, j: (i, 0, 0)),
            pl.BlockSpec((1, 1, HID), lambda i, j: (i, 0, 0)),
            pl.BlockSpec((1, 1, 1), lambda i, j: (i, 0, 0)),
            pl.BlockSpec((DH, HID), lambda i, j: (0, 0)),
        ],
        out_specs=pl.BlockSpec((1, _ROWB, HID), lambda i, j: (i, j, 0)),
    )(acc_stack, e_stack, x_stack, w_stack, b_stack, gate_stack, s_const)


_POS = (0, 8, 4, 12, 2, 10, 6, 14)
_HEAD_OF_LANE = (0, 0, 4, 4, 2, 2, 6, 6, 1, 1, 5, 5, 3, 3, 7, 7)
_DNUMS = lax.GatherDimensionNumbers(
    offset_dims=(), collapsed_slice_dims=(0,), start_index_map=(0,))


def _perm(v, idx):
    return lax.gather(v, idx[:, None], _DNUMS, (1,),
                      mode=lax.GatherScatterMode.PROMISE_IN_BOUNDS)


def _combine(a, b, k, lane):
    m = (lane & k) == 0
    return jnp.where(m, a, _perm(b, lane ^ k)) + jnp.where(m, _perm(a, lane ^ k), b)


def _edge_phase(sid, kk_hbm, q_hbm, vv_hbm, si_hbm, di_hbm, zr_hbm, out_hbm,
                oute_hbm, acc_ew, acc_e, bufs, sems):
    row_sl = pl.ds(sid * RPT, RPT)
    pltpu.sync_copy(zr_hbm, acc_ew.at[row_sl])
    pltpu.sync_copy(zr_hbm.at[:, pl.ds(0, DH)], acc_e.at[row_sl])
    plsc.subcore_barrier()

    lane = lax.iota(jnp.int32, DH)
    base = sid * EPT

    def load_idx(b, off):
        si_v, di_v = bufs[b][0], bufs[b][1]
        pltpu.sync_copy(si_hbm.at[pl.ds(off, BATCH)], si_v)
        pltpu.sync_copy(di_hbm.at[pl.ds(off, BATCH)], di_v)

    def start_gathers(b):
        si_v, di_v, kk_v, q_v, vv_v, pe_v = bufs[b]
        pltpu.async_copy(kk_hbm.at[si_v], kk_v, sems[b][0])
        pltpu.async_copy(q_hbm.at[di_v], q_v, sems[b][1])
        pltpu.async_copy(vv_hbm.at[si_v], vv_v, sems[b][2])

    def wait_gathers(b):
        si_v, di_v, kk_v, q_v, vv_v, pe_v = bufs[b]
        pltpu.make_async_copy(kk_hbm.at[si_v], kk_v, sems[b][0]).wait()
        pltpu.make_async_copy(q_hbm.at[di_v], q_v, sems[b][1]).wait()
        pltpu.make_async_copy(vv_hbm.at[si_v], vv_v, sems[b][2]).wait()

    def wait_scatters(b):
        si_v, di_v, kk_v, q_v, vv_v, pe_v = bufs[b]
        pltpu.make_async_copy(vv_v, acc_ew.at[di_v], sems[b][3]).wait()
        pltpu.make_async_copy(pe_v, acc_e.at[di_v], sems[b][4]).wait()

    def compute_scatter(b):
        si_v, di_v, kk_v, q_v, vv_v, pe_v = bufs[b]
        wait_gathers(b)

        def step_body(j, carry2):
            for u in range(UNROLL):
                e = j * UNROLL + u
                prods = [kk_v[e, pl.ds(h * DH, DH)] * q_v[e, pl.ds(h * DH, DH)]
                         for h in range(HEADS)]
                l1 = [_combine(prods[2 * i], prods[2 * i + 1], 8, lane)
                      for i in range(4)]
                l2 = [_combine(l1[2 * i], l1[2 * i + 1], 4, lane)
                      for i in range(2)]
                l3 = _combine(l2[0], l2[1], 2, lane)
                r = l3 + _perm(l3, lane ^ 1)
                pe = jnp.exp(r)
                pe_v[e, :] = pe
                for h in range(HEADS):
                    sl = pl.ds(h * DH, DH)
                    esp = _perm(pe, jnp.full((DH,), _POS[h], jnp.int32))
                    vv_v[e, sl] = vv_v[e, sl] * esp
            return carry2

        lax.fori_loop(0, BATCH // UNROLL, step_body, 0)
        pltpu.async_copy(vv_v, acc_ew.at[di_v], sems[b][3], add=True)
        pltpu.async_copy(pe_v, acc_e.at[di_v], sems[b][4], add=True)

    load_idx(0, base)
    start_gathers(0)

    def pair_body(i, carry):
        off = base + (2 * i) * BATCH

        @pl.when(i > 0)
        def _():
            wait_scatters(1)

        load_idx(1, off + BATCH)
        start_gathers(1)
        compute_scatter(0)

        @pl.when(i < NBATCH // 2 - 1)
        def _():
            wait_scatters(0)
            load_idx(0, off + 2 * BATCH)
            start_gathers(0)

        compute_scatter(1)
        return carry

    lax.fori_loop(0, NBATCH // 2, pair_body, 0)
    wait_scatters(0)
    wait_scatters(1)
    plsc.subcore_barrier()
    pltpu.sync_copy(acc_ew.at[row_sl], out_hbm.at[row_sl])
    pltpu.sync_copy(acc_e.at[row_sl], oute_hbm.at[row_sl])


def _sc_body(kk_a, q_a, vv_a, si_a, di_a,
             kk_b, q_b, vv_b, si_b, di_b, zr,
             out_a, oute_a, out_b, oute_b,
             acc_ew, acc_e,
             si0, di0, kk0, q0, vv0, pe0, si1, di1, kk1, q1, vv1, pe1,
             s00, s01, s02, s03, s04, s10, s11, s12, s13, s14):
    cid = lax.axis_index("c")
    sid = lax.axis_index("s")
    bufs = ((si0, di0, kk0, q0, vv0, pe0), (si1, di1, kk1, q1, vv1, pe1))
    sems = ((s00, s01, s02, s03, s04), (s10, s11, s12, s13, s14))

    @pl.when(cid == 0)
    def _():
        _edge_phase(sid, kk_a, q_a, vv_a, si_a, di_a, zr, out_a, oute_a,
                    acc_ew, acc_e, bufs, sems)

    @pl.when(cid == 1)
    def _():
        _edge_phase(sid, kk_b, q_b, vv_b, si_b, di_b, zr, out_b, oute_b,
                    acc_ew, acc_e, bufs, sems)


@functools.lru_cache(maxsize=1)
def _build_sc_kernel():
    return pl.kernel(
        _sc_body,
        out_type=(
            jax.ShapeDtypeStruct((NPAD, HID), jnp.float32),
            jax.ShapeDtypeStruct((NPAD, DH), jnp.float32),
            jax.ShapeDtypeStruct((NPAD, HID), jnp.float32),
            jax.ShapeDtypeStruct((NPAD, DH), jnp.float32),
        ),
        mesh=plsc.VectorSubcoreMesh(
            core_axis_name="c", subcore_axis_name="s",
            num_cores=2, num_subcores=NTILES),
        scratch_types=[
            pltpu.VMEM_SHARED((NPAD, HID), jnp.float32),
            pltpu.VMEM_SHARED((NPAD, DH), jnp.float32),
            pltpu.VMEM((BATCH,), jnp.int32),
            pltpu.VMEM((BATCH,), jnp.int32),
            pltpu.VMEM((BATCH, HID), jnp.float32),
            pltpu.VMEM((BATCH, HID), jnp.float32),
            pltpu.VMEM((BATCH, HID), jnp.float32),
            pltpu.VMEM((BATCH, DH), jnp.float32),
            pltpu.VMEM((BATCH,), jnp.int32),
            pltpu.VMEM((BATCH,), jnp.int32),
            pltpu.VMEM((BATCH, HID), jnp.float32),
            pltpu.VMEM((BATCH, HID), jnp.float32),
            pltpu.VMEM((BATCH, HID), jnp.float32),
            pltpu.VMEM((BATCH, DH), jnp.float32),
            pltpu.SemaphoreType.DMA,
            pltpu.SemaphoreType.DMA,
            pltpu.SemaphoreType.DMA,
            pltpu.SemaphoreType.DMA,
            pltpu.SemaphoreType.DMA,
            pltpu.SemaphoreType.DMA,
            pltpu.SemaphoreType.DMA,
            pltpu.SemaphoreType.DMA,
            pltpu.SemaphoreType.DMA,
            pltpu.SemaphoreType.DMA,
        ],
        compiler_params=pltpu.CompilerParams(
            use_tc_tiling_on_sc=False, needs_layout_passes=False),
    )


def _sc_edges(*args):
    return _build_sc_kernel()(*args)


def _block_diag8(rel):
    out = jnp.zeros((HID, HID), jnp.float32)
    for h in range(HEADS):
        out = out.at[h * DH:(h + 1) * DH, h * DH:(h + 1) * DH].set(rel[h])
    return out


def kernel(x_user, x_item, edge_index_user_item, edge_index_item_user, params):
    ei_a = edge_index_user_item.astype(jnp.int32)
    ei_b = edge_index_item_user.astype(jnp.int32)
    si_a, di_a = ei_a[0], ei_a[1]
    si_b, di_b = ei_b[0], ei_b[1]

    s_const = jnp.zeros((DH, HID), jnp.float32)
    for l in range(DH):
        h = _HEAD_OF_LANE[l]
        s_const = s_const.at[l, h * DH:(h + 1) * DH].set(0.5)
    zr = jnp.zeros((RPT, HID), jnp.float32)

    x = jnp.stack([x_user, x_item])
    ek = {"user": "user__to__item", "item": "item__to__user"}

    for p in params:
        w_list, b_list, wo_list, bo_list, g_list = [], [], [], [], []
        for t in ("user", "item"):
            rel = ek[t]
            a_bd = _block_diag8(
                p["a_rel"][rel] * (p["p_rel"][rel] / jnp.sqrt(float(DH)))[:, None, None])
            m_bd = _block_diag8(p["m_rel"][rel])
            wk = p["k"][t]["w"] @ a_bd
            bk = p["k"][t]["b"] @ a_bd
            wv = p["v"][t]["w"] @ m_bd
            bv = p["v"][t]["b"] @ m_bd
            w_list.append(jnp.concatenate(
                [wk, p["q"][t]["w"], wv], axis=1))
            b_list.append(jnp.concatenate(
                [bk, p["q"][t]["b"], bv])[None])
            wo_list.append(p["out"][t]["w"])
            bo_list.append(p["out"][t]["b"][None])
            g_list.append(jnp.full((1, 1), p["skip"][t]))

        proj = _proj(x, jnp.stack(w_list), jnp.stack(b_list))
        kk = proj[:, :, :HID]
        q = proj[:, :, HID:2 * HID]
        vv = proj[:, :, 2 * HID:]

        ew_item, e_item, ew_user, e_user = _sc_edges(
            kk[0], q[1], vv[0], si_a, di_a,
            kk[1], q[0], vv[1], si_b, di_b, zr)

        x = _out_proj(
            jnp.stack([ew_user[:NNODE], ew_item[:NNODE]]),
            jnp.stack([e_user[:NNODE], e_item[:NNODE]]), x,
            jnp.stack(wo_list), jnp.stack(bo_list), jnp.stack(g_list),
            s_const)

    return (x[0], x[1])
